# speedup vs baseline: 1.2958x; 1.2958x over previous
import functools

import jax
import jax.numpy as jnp
from jax import lax
from jax.experimental import pallas as pl
from jax.experimental.pallas import tpu as pltpu
from jax.experimental.pallas import tpu_sc as plsc

N = 160000
D = 256
NUM_RELS = 1000
NC = 2
NS = 16
NW = NC * NS
PER_W = N // NW
B = 40
NBLK = PER_W // B
LANES = 16
SL = D // LANES


def _mesh():
    return plsc.VectorSubcoreMesh(core_axis_name="c", subcore_axis_name="s")


@functools.partial(
    pl.kernel,
    mesh=_mesh(),
    out_type=jax.ShapeDtypeStruct((N, D), jnp.float32),
    scratch_types=[
        pltpu.VMEM((B,), jnp.int32),
        pltpu.VMEM((B, D), jnp.float32),
        pltpu.VMEM((B, D), jnp.float32),
        pltpu.SemaphoreType.DMA,
        pltpu.SemaphoreType.DMA,
    ],
)
def _tail_sc(head_hbm, idx_hbm, w_hbm, out_hbm, idx_v, rel_v, head_v, gsem, hsem):
    wid = lax.axis_index("s") * NC + lax.axis_index("c")

    def blk_body(blk, carry):
        base = wid * PER_W + blk * B
        pltpu.sync_copy(idx_hbm.at[pl.ds(base, B)], idx_v)
        gcopy = pltpu.async_copy(w_hbm.at[idx_v], rel_v, gsem)
        hcopy = pltpu.async_copy(head_hbm.at[pl.ds(base, B), :], head_v, hsem)
        hcopy.wait()
        gcopy.wait()

        def row_body(i, c):
            for j in range(SL):
                sl = pl.ds(j * LANES, LANES)
                rel_v[i, sl] = rel_v[i, sl] + head_v[i, sl]
            return c

        lax.fori_loop(0, B, row_body, 0)
        pltpu.sync_copy(rel_v, out_hbm.at[pl.ds(base, B), :])
        return carry

    lax.fori_loop(0, NBLK, blk_body, 0)


def kernel(head, rel_idx, w_relation):
    return _tail_sc(head, rel_idx.astype(jnp.int32), w_relation)


# speedup vs baseline: 2.5160x; 1.9417x over previous
import functools

import jax
import jax.numpy as jnp
from jax import lax
from jax.experimental import pallas as pl
from jax.experimental.pallas import tpu as pltpu
from jax.experimental.pallas import tpu_sc as plsc

N = 160000
D = 256
NC = 2
NS = 16
NW = NC * NS
PER_W = N // NW
B = 40
NBLK = PER_W // B
LANES = 16
SL = D // LANES
NBUF = 2


def _mesh():
    return plsc.VectorSubcoreMesh(core_axis_name="c", subcore_axis_name="s")


@functools.partial(
    pl.kernel,
    mesh=_mesh(),
    out_type=jax.ShapeDtypeStruct((N, D), jnp.float32),
    scratch_types=[
        pltpu.VMEM((PER_W,), jnp.int32),
        pltpu.VMEM((NBUF, B, D), jnp.float32),
        pltpu.VMEM((NBUF, B, D), jnp.float32),
        pltpu.VMEM((NBUF, B, D), jnp.float32),
        pltpu.SemaphoreType.DMA,
        pltpu.SemaphoreType.DMA,
        pltpu.SemaphoreType.DMA,
        pltpu.SemaphoreType.DMA,
        pltpu.SemaphoreType.DMA,
        pltpu.SemaphoreType.DMA,
    ],
)
def _tail_sc(head_hbm, idx_hbm, w_hbm, out_hbm,
             idx_v, rel_v, head_v, out_v, g0, g1, h0, h1, o0, o1):
    wid = lax.axis_index("s") * NC + lax.axis_index("c")
    wbase = wid * PER_W
    gsems = (g0, g1)
    hsems = (h0, h1)
    osems = (o0, o1)

    pltpu.sync_copy(idx_hbm.at[pl.ds(wbase, PER_W)], idx_v)

    def issue_loads(blk, s):
        base = wbase + blk * B
        pltpu.async_copy(
            w_hbm.at[idx_v.at[pl.ds(blk * B, B)]], rel_v.at[s], gsems[s])
        pltpu.async_copy(
            head_hbm.at[pl.ds(base, B), :], head_v.at[s], hsems[s])

    for s in range(NBUF):
        issue_loads(s, s)

    def outer(t, carry):
        for s in range(NBUF):
            blk = t * NBUF + s
            base = wbase + blk * B

            @pl.when(blk < NBLK)
            def _body():
                pltpu.make_async_copy(
                    w_hbm.at[idx_v.at[pl.ds(blk * B, B)]],
                    rel_v.at[s], gsems[s]).wait()
                pltpu.make_async_copy(
                    head_hbm.at[pl.ds(base, B), :],
                    head_v.at[s], hsems[s]).wait()

                @pl.when(blk >= NBUF)
                def _drain():
                    pltpu.make_async_copy(
                        out_v.at[s], out_hbm.at[pl.ds(base, B), :],
                        osems[s]).wait()

                rv = rel_v.at[s]
                hv = head_v.at[s]
                ov = out_v.at[s]

                def row_body(i, c):
                    for j in range(SL):
                        sl = pl.ds(j * LANES, LANES)
                        ov[i, sl] = rv[i, sl] + hv[i, sl]
                    return c

                lax.fori_loop(0, B, row_body, 0)

                @pl.when(blk + NBUF < NBLK)
                def _refill():
                    issue_loads(blk + NBUF, s)

                pltpu.async_copy(
                    ov, out_hbm.at[pl.ds(base, B), :], osems[s])

        return carry

    lax.fori_loop(0, (NBLK + NBUF - 1) // NBUF, outer, 0)

    for s in range(NBUF):
        pltpu.make_async_copy(
            out_v.at[s], out_hbm.at[pl.ds(wbase, B), :], osems[s]).wait()


def kernel(head, rel_idx, w_relation):
    return _tail_sc(head, rel_idx.astype(jnp.int32), w_relation)


# speedup vs baseline: 2.5208x; 1.0019x over previous
import functools

import jax
import jax.numpy as jnp
from jax import lax
from jax.experimental import pallas as pl
from jax.experimental.pallas import tpu as pltpu
from jax.experimental.pallas import tpu_sc as plsc

N = 160000
D = 256
NC = 2
NS = 16
NW = NC * NS
PER_W = N // NW
B = 40
NBLK = PER_W // B
LANES = 16
SL = D // LANES
NBUF = 3


def _mesh():
    return plsc.VectorSubcoreMesh(core_axis_name="c", subcore_axis_name="s")


@functools.partial(
    pl.kernel,
    mesh=_mesh(),
    out_type=jax.ShapeDtypeStruct((N, D), jnp.float32),
    scratch_types=[
        pltpu.VMEM((PER_W,), jnp.int32),
        pltpu.VMEM((NBUF, B, D), jnp.float32),
        pltpu.VMEM((NBUF, B, D), jnp.float32),
        pltpu.VMEM((NBUF, B, D), jnp.float32),
        pltpu.SemaphoreType.DMA,
        pltpu.SemaphoreType.DMA,
        pltpu.SemaphoreType.DMA,
        pltpu.SemaphoreType.DMA,
        pltpu.SemaphoreType.DMA,
        pltpu.SemaphoreType.DMA,
        pltpu.SemaphoreType.DMA,
        pltpu.SemaphoreType.DMA,
        pltpu.SemaphoreType.DMA,
    ],
)
def _tail_sc(head_hbm, idx_hbm, w_hbm, out_hbm,
             idx_v, rel_v, head_v, out_v,
             g0, g1, g2, h0, h1, h2, o0, o1, o2):
    wid = lax.axis_index("s") * NC + lax.axis_index("c")
    wbase = wid * PER_W
    gsems = (g0, g1, g2)
    hsems = (h0, h1, h2)
    osems = (o0, o1, o2)

    pltpu.sync_copy(idx_hbm.at[pl.ds(wbase, PER_W)], idx_v)

    def issue_loads(blk, s):
        base = wbase + blk * B
        pltpu.async_copy(
            w_hbm.at[idx_v.at[pl.ds(blk * B, B)]], rel_v.at[s], gsems[s])
        pltpu.async_copy(
            head_hbm.at[pl.ds(base, B), :], head_v.at[s], hsems[s])

    for s in range(NBUF):
        issue_loads(s, s)

    def outer(t, carry):
        for s in range(NBUF):
            blk = t * NBUF + s
            base = wbase + blk * B

            @pl.when(blk < NBLK)
            def _body():
                pltpu.make_async_copy(
                    w_hbm.at[idx_v.at[pl.ds(blk * B, B)]],
                    rel_v.at[s], gsems[s]).wait()
                pltpu.make_async_copy(
                    head_hbm.at[pl.ds(base, B), :],
                    head_v.at[s], hsems[s]).wait()

                @pl.when(blk >= NBUF)
                def _drain():
                    pltpu.make_async_copy(
                        out_v.at[s], out_hbm.at[pl.ds(base, B), :],
                        osems[s]).wait()

                rv = rel_v.at[s]
                hv = head_v.at[s]
                ov = out_v.at[s]

                def row_body(i, c):
                    for j in range(SL):
                        sl = pl.ds(j * LANES, LANES)
                        ov[i, sl] = rv[i, sl] + hv[i, sl]
                    return c

                lax.fori_loop(0, B, row_body, 0)

                @pl.when(blk + NBUF < NBLK)
                def _refill():
                    issue_loads(blk + NBUF, s)

                pltpu.async_copy(
                    ov, out_hbm.at[pl.ds(base, B), :], osems[s])

        return carry

    lax.fori_loop(0, (NBLK + NBUF - 1) // NBUF, outer, 0)

    for s in range(NBUF):
        pltpu.make_async_copy(
            out_v.at[s], out_hbm.at[pl.ds(wbase, B), :], osems[s]).wait()


def kernel(head, rel_idx, w_relation):
    return _tail_sc(head, rel_idx.astype(jnp.int32), w_relation)
